# baseline (speedup 1.0000x reference)
import functools

import jax
import jax.numpy as jnp
from jax import lax
from jax.experimental import pallas as pl
from jax.experimental.pallas import tpu as pltpu
from jax.experimental.pallas import tpu_sc as plsc

VOCAB = 1000000
HID = 64
MAXLEN = 200
N = 4096 * 200
NC = 2
NS = 16
NW = NC * NS
PER_W = N // NW
CH = 512
STEPS = CH // 128
N_CHUNKS = PER_W // CH


def _body(xf_hbm, pf_hbm, wword_hbm, wpos_hbm, out_hbm,
          xidx_v, pidx_v, rows_v, sem, sem2):
    wid = lax.axis_index("s") * NC + lax.axis_index("c")
    base = wid * PER_W

    def chunk_body(c, _):
        start = base + c * CH
        xrow = pl.multiple_of(start // 128, STEPS)

        pltpu.sync_copy(xf_hbm.at[pl.ds(xrow, STEPS)], xidx_v)
        pltpu.sync_copy(pf_hbm.at[pl.ds(xrow, STEPS)], pidx_v)

        cps = [
            pltpu.async_copy(
                wword_hbm.at[xidx_v.at[s]],
                rows_v.at[pl.ds(s * 128, 128)],
                sem,
            )
            for s in range(STEPS)
        ]
        for cp in cps:
            cp.wait()

        cps2 = [
            pltpu.async_copy(
                wpos_hbm.at[pidx_v.at[s]],
                rows_v.at[pl.ds(s * 128, 128)],
                sem2,
                add=True,
            )
            for s in range(STEPS)
        ]
        for cp in cps2:
            cp.wait()

        pltpu.sync_copy(rows_v, out_hbm.at[pl.ds(start, CH)])
        return 0

    lax.fori_loop(0, N_CHUNKS, chunk_body, 0)


@jax.jit
def _emb(xf, pf, wword, wpos):
    mesh = plsc.VectorSubcoreMesh(core_axis_name="c", subcore_axis_name="s")
    f = functools.partial(
        pl.kernel,
        out_type=jax.ShapeDtypeStruct((N, HID), jnp.float32),
        mesh=mesh,
        compiler_params=pltpu.CompilerParams(
            needs_layout_passes=False, use_tc_tiling_on_sc=False),
        scratch_types=[
            pltpu.VMEM((STEPS, 128), jnp.int32),
            pltpu.VMEM((STEPS, 128), jnp.int32),
            pltpu.VMEM((CH, HID), jnp.float32),
            pltpu.SemaphoreType.DMA,
            pltpu.SemaphoreType.DMA,
        ],
    )(_body)
    return f(xf, pf, wword, wpos)


def kernel(X, pos, W_word, W_pos):
    xf = X.reshape(N // 128, 128).astype(jnp.int32)
    pf = pos.reshape(N // 128, 128).astype(jnp.int32)
    out = _emb(xf, pf, W_word, W_pos)
    return out.reshape(X.shape + (HID,))
